# baseline (speedup 1.0000x reference)
import functools

import jax
import jax.numpy as jnp
from jax import lax
from jax.experimental import pallas as pl
from jax.experimental.pallas import tpu as pltpu
from jax.experimental.pallas import tpu_sc as plsc

DIM = 64
_RPC = 8
_BN = 8192


def _tc_table(U, sigma, Vt, rv):
    V = U.shape[0]
    Ut = U.T
    grid = (pl.cdiv(V, _BN),)

    def table_k(ut_ref, s_ref, vt_ref, rv_ref, o_ref):
        M = s_ref[...] * vt_ref[...]
        for i in range(rv.shape[0]):
            v = rv_ref[i:i + 1, :]
            vn = v / (jnp.sqrt(jnp.sum(v * v)) + 1e-12)
            Mv = jnp.sum(M * vn, axis=1, keepdims=True)
            M = M + Mv * v
        o_ref[...] = lax.dot_general(
            ut_ref[...], M, (((0,), (0,)), ((), ())),
            preferred_element_type=jnp.float32,
        )

    return pl.pallas_call(
        table_k,
        grid=grid,
        in_specs=[
            pl.BlockSpec((DIM, _BN), lambda i: (0, i)),
            pl.BlockSpec((DIM, 1), lambda i: (0, 0)),
            pl.BlockSpec((DIM, DIM), lambda i: (0, 0)),
            pl.BlockSpec((rv.shape[0], DIM), lambda i: (0, 0)),
        ],
        out_specs=pl.BlockSpec((_BN, DIM), lambda i: (i, 0)),
        out_shape=jax.ShapeDtypeStruct((V, DIM), jnp.float32),
    )(Ut, sigma.reshape(DIM, 1), Vt, rv)


def _sc_gather(table, indices):
    NR, L = indices.shape
    B = NR * L
    _p1 = min(128, -(-(L // 2) // 8) * 8)
    parts = ((0, _p1), (_p1, L - _p1))
    info = plsc.get_sparse_core_info()
    NC, NS = info.num_cores, info.num_subcores
    NW = NC * NS
    r_per_w = NR // NW
    n_ch = r_per_w // _RPC
    mesh = plsc.VectorSubcoreMesh(core_axis_name="c", subcore_axis_name="s")

    @functools.partial(
        pl.kernel,
        out_type=jax.ShapeDtypeStruct((B, DIM), jnp.float32),
        mesh=mesh,
        compiler_params=pltpu.CompilerParams(use_tc_tiling_on_sc=False),
        scratch_types=[
            pltpu.VMEM((_RPC, L), jnp.int32),
            pltpu.VMEM((_RPC * L, DIM), jnp.float32),
            pltpu.SemaphoreType.DMA,
        ],
    )
    def gather_k(table_hbm, idx_hbm, out_hbm, idx_v, rows_v, gsem):
        wid = lax.axis_index("s") * NC + lax.axis_index("c")
        row_base = wid * r_per_w

        def chunk(g, _):
            r0 = row_base + g * _RPC
            pltpu.sync_copy(idx_hbm.at[pl.ds(r0, _RPC)], idx_v)
            for r in range(_RPC):
                for off, sz in parts:
                    pltpu.async_copy(
                        table_hbm.at[idx_v.at[r, pl.ds(off, sz)]],
                        rows_v.at[pl.ds(r * L + off, sz)],
                        gsem,
                    )
            for r in range(_RPC):
                for off, sz in parts:
                    pltpu.make_async_copy(
                        table_hbm.at[idx_v.at[r, pl.ds(off, sz)]],
                        rows_v.at[pl.ds(r * L + off, sz)],
                        gsem,
                    ).wait()
            pltpu.sync_copy(rows_v, out_hbm.at[pl.ds(r0 * L, _RPC * L)])
            return _

        lax.fori_loop(0, n_ch, chunk, None)

    return gather_k(table, indices)


def kernel(indices, U, sigma, Vt, right_vecs):
    NR, L = indices.shape
    T = _tc_table(U, sigma, Vt, right_vecs)
    y = _sc_gather(T, indices.astype(jnp.int32))
    return y.reshape(NR, L, DIM)


# speedup vs baseline: 1.0384x; 1.0384x over previous
import functools

import jax
import jax.numpy as jnp
from jax import lax
from jax.experimental import pallas as pl
from jax.experimental.pallas import tpu as pltpu
from jax.experimental.pallas import tpu_sc as plsc

DIM = 64
_RPC = 8
_BN = 32768


def _tc_table(U, sigma, Vt, rv):
    V = U.shape[0]
    Ut = U.T
    grid = (pl.cdiv(V, _BN),)

    def table_k(ut_ref, s_ref, vt_ref, rv_ref, o_ref):
        M = s_ref[...] * vt_ref[...]
        for i in range(rv.shape[0]):
            v = rv_ref[i:i + 1, :]
            vn = v / (jnp.sqrt(jnp.sum(v * v)) + 1e-12)
            Mv = jnp.sum(M * vn, axis=1, keepdims=True)
            M = M + Mv * v
        o_ref[...] = lax.dot_general(
            ut_ref[...], M, (((0,), (0,)), ((), ())),
            preferred_element_type=jnp.float32,
        )

    return pl.pallas_call(
        table_k,
        grid=grid,
        in_specs=[
            pl.BlockSpec((DIM, _BN), lambda i: (0, i)),
            pl.BlockSpec((DIM, 1), lambda i: (0, 0)),
            pl.BlockSpec((DIM, DIM), lambda i: (0, 0)),
            pl.BlockSpec((rv.shape[0], DIM), lambda i: (0, 0)),
        ],
        out_specs=pl.BlockSpec((_BN, DIM), lambda i: (i, 0)),
        out_shape=jax.ShapeDtypeStruct((V, DIM), jnp.float32),
    )(Ut, sigma.reshape(DIM, 1), Vt, rv)


def _sc_gather(table, indices):
    NR, L = indices.shape
    B = NR * L
    _p1 = min(128, -(-(L // 2) // 8) * 8)
    parts = ((0, _p1), (_p1, L - _p1))
    info = plsc.get_sparse_core_info()
    NC, NS = info.num_cores, info.num_subcores
    NW = NC * NS
    r_per_w = NR // NW
    n_ch = r_per_w // _RPC
    mesh = plsc.VectorSubcoreMesh(core_axis_name="c", subcore_axis_name="s")

    @functools.partial(
        pl.kernel,
        out_type=jax.ShapeDtypeStruct((B, DIM), jnp.float32),
        mesh=mesh,
        compiler_params=pltpu.CompilerParams(use_tc_tiling_on_sc=False),
        scratch_types=[
            pltpu.VMEM((_RPC, L), jnp.int32),
            pltpu.VMEM((_RPC * L, DIM), jnp.float32),
            pltpu.SemaphoreType.DMA,
        ],
    )
    def gather_k(table_hbm, idx_hbm, out_hbm, idx_v, rows_v, gsem):
        wid = lax.axis_index("s") * NC + lax.axis_index("c")
        row_base = wid * r_per_w

        def chunk(g, _):
            r0 = row_base + g * _RPC
            pltpu.sync_copy(idx_hbm.at[pl.ds(r0, _RPC)], idx_v)
            for r in range(_RPC):
                for off, sz in parts:
                    pltpu.async_copy(
                        table_hbm.at[idx_v.at[r, pl.ds(off, sz)]],
                        rows_v.at[pl.ds(r * L + off, sz)],
                        gsem,
                    )
            for r in range(_RPC):
                for off, sz in parts:
                    pltpu.make_async_copy(
                        table_hbm.at[idx_v.at[r, pl.ds(off, sz)]],
                        rows_v.at[pl.ds(r * L + off, sz)],
                        gsem,
                    ).wait()
            pltpu.sync_copy(rows_v, out_hbm.at[pl.ds(r0 * L, _RPC * L)])
            return _

        lax.fori_loop(0, n_ch, chunk, None)

    return gather_k(table, indices)


def kernel(indices, U, sigma, Vt, right_vecs):
    NR, L = indices.shape
    T = _tc_table(U, sigma, Vt, right_vecs)
    y = _sc_gather(T, indices.astype(jnp.int32))
    return y.reshape(NR, L, DIM)


# speedup vs baseline: 1.1785x; 1.1349x over previous
import functools

import jax
import jax.numpy as jnp
from jax import lax
from jax.experimental import pallas as pl
from jax.experimental.pallas import tpu as pltpu
from jax.experimental.pallas import tpu_sc as plsc

DIM = 64
_RPC = 8


def _sc_gather(table, indices):
    NR, L = indices.shape
    B = NR * L
    _p1 = min(128, -(-(L // 2) // 8) * 8)
    parts = ((0, _p1), (_p1, L - _p1))
    info = plsc.get_sparse_core_info()
    NC, NS = info.num_cores, info.num_subcores
    NW = NC * NS
    r_per_w = NR // NW
    n_ch = r_per_w // _RPC
    half = n_ch // 2
    rows_pw = r_per_w * L
    mesh = plsc.VectorSubcoreMesh(core_axis_name="c", subcore_axis_name="s")

    @functools.partial(
        pl.kernel,
        out_type=jax.ShapeDtypeStruct((B // 2, 128), jnp.float32),
        mesh=mesh,
        compiler_params=pltpu.CompilerParams(use_tc_tiling_on_sc=False),
        scratch_types=[
            pltpu.VMEM((_RPC, L), jnp.int32),
            pltpu.VMEM((_RPC * L, DIM), jnp.float32),
            pltpu.SemaphoreType.DMA,
        ],
    )
    def gather_k(table_hbm, idx_hbm, out_hbm, idx_v, rows_v, gsem):
        wid = lax.axis_index("s") * NC + lax.axis_index("c")
        row_base = wid * r_per_w
        out_base = wid * (rows_pw // 2)

        def chunk(g, lane_off):
            r0 = row_base + g * _RPC
            pltpu.sync_copy(idx_hbm.at[pl.ds(r0, _RPC)], idx_v)
            for r in range(_RPC):
                for off, sz in parts:
                    pltpu.async_copy(
                        table_hbm.at[idx_v.at[r, pl.ds(off, sz)]],
                        rows_v.at[pl.ds(r * L + off, sz)],
                        gsem,
                    )
            for r in range(_RPC):
                for off, sz in parts:
                    pltpu.make_async_copy(
                        table_hbm.at[idx_v.at[r, pl.ds(off, sz)]],
                        rows_v.at[pl.ds(r * L + off, sz)],
                        gsem,
                    ).wait()
            pltpu.sync_copy(
                rows_v,
                out_hbm.at[
                    pl.ds(out_base + (g % half) * (_RPC * L), _RPC * L),
                    pl.ds(lane_off, DIM),
                ],
            )

        def body_lo(g, _):
            chunk(g, 0)
            return _

        def body_hi(g, _):
            chunk(g, DIM)
            return _

        lax.fori_loop(0, half, body_lo, None)
        lax.fori_loop(half, n_ch, body_hi, None)

    return gather_k(table, indices), rows_pw


def _tc_project(x, sigma, Vt, rv, NR, L, rows_pw):
    B = NR * L
    BT = rows_pw // 2
    grid = (B // 2) // BT

    def proj_k(x_ref, s_ref, vt_ref, rv_ref, o_ref):
        M = s_ref[...] * vt_ref[...]
        for i in range(rv.shape[0]):
            v = rv_ref[i:i + 1, :]
            vn = v / (jnp.sqrt(jnp.sum(v * v)) + 1e-12)
            Mv = jnp.sum(M * vn, axis=1, keepdims=True)
            M = M + Mv * v
        xp = x_ref[...]
        z = jnp.concatenate([xp[:, :DIM], xp[:, DIM:]], axis=0)
        o_ref[...] = jnp.dot(z, M, preferred_element_type=jnp.float32)

    y = pl.pallas_call(
        proj_k,
        grid=(grid,),
        in_specs=[
            pl.BlockSpec((BT, 128), lambda i: (i, 0)),
            pl.BlockSpec((DIM, 1), lambda i: (0, 0)),
            pl.BlockSpec((DIM, DIM), lambda i: (0, 0)),
            pl.BlockSpec((rv.shape[0], DIM), lambda i: (0, 0)),
        ],
        out_specs=pl.BlockSpec((2 * BT, DIM), lambda i: (i, 0)),
        out_shape=jax.ShapeDtypeStruct((B, DIM), jnp.float32),
    )(x, sigma.reshape(DIM, 1), Vt, rv)
    return y.reshape(NR, L, DIM)


def kernel(indices, U, sigma, Vt, right_vecs):
    NR, L = indices.shape
    gathered, rows_pw = _sc_gather(U, indices.astype(jnp.int32))
    return _tc_project(gathered, sigma, Vt, right_vecs, NR, L, rows_pw)
